# Initial kernel scaffold; baseline (speedup 1.0000x reference)
import jax
import jax.numpy as jnp
from jax.experimental import pallas as pl


def kernel(x, edge_index, batch, emb, Wq, bq, Wk, bk, Wv, bv, Wskip, bskip):
    raise NotImplementedError("write your pallas kernel here")


# speedup vs baseline: 14.4317x; 14.4317x over previous
import functools
import math

import jax
import jax.numpy as jnp
from jax import lax
from jax.experimental import pallas as pl
from jax.experimental.pallas import tpu as pltpu
from jax.experimental.pallas import tpu_sc as plsc

N_NODES = 10000
N_EDGES = 320000
F_IN = 128
HEADS = 4
CH = 32
N_LAYERS = 4
VIRT = 4
N_GRAPHS = 8

D = HEADS * CH
N_EXT = N_NODES + VIRT * N_GRAPHS
N_PAD = 10240
DUMMY = N_PAD - 1
E_TOT = N_EDGES + (N_EXT * VIRT) + N_NODES
NW = 32
C = 64
NCH = -(-E_TOT // (NW * C))
E_PAD = NW * C * NCH
C2 = 512
NCH2 = E_PAD // C2
INV_SQRT_CH = 1.0 / math.sqrt(float(CH))
R_TC = 256
_GRID = N_PAD // R_TC


def _split_proj(y, vt, q_ref, k_ref, s_ref, vt_ref):
    q_ref[...] = y[:, 0:D]
    k_ref[...] = y[:, D:2 * D]
    s_ref[...] = y[:, 3 * D:4 * D]
    vt_ref[...] = vt


def _proj_first_body(h_ref, w_ref, b_ref, bt_ref, q_ref, k_ref, s_ref, vt_ref):
    hh = h_ref[...]
    y = jnp.dot(hh, w_ref[...], preferred_element_type=jnp.float32)
    y = y + b_ref[0:1, :]
    vt = lax.dot_general(w_ref[:, 2 * D:3 * D], hh,
                         (((0,), (1,)), ((), ())),
                         preferred_element_type=jnp.float32)
    vt = vt + bt_ref[2 * D:3 * D, 0:1]
    _split_proj(y, vt, q_ref, k_ref, s_ref, vt_ref)


def _assemble(o_ref, d_ref, sk_ref):
    o = jnp.concatenate([o_ref[wi] for wi in range(NW)], axis=1)
    d = jnp.sum(d_ref[...], axis=0) + 1e-16
    parts = []
    for h in range(HEADS):
        dh = jnp.broadcast_to(d[:, h:h + 1], (R_TC, CH))
        parts.append(o[:, h * CH:(h + 1) * CH] / dh)
    return jnp.concatenate(parts, axis=1) + sk_ref[...]


def _proj_mid_body(o_ref, d_ref, sk_ref, w_ref, b_ref, bt_ref,
                   q_ref, k_ref, s_ref, vt_ref):
    hh = _assemble(o_ref, d_ref, sk_ref)
    y = jnp.dot(hh, w_ref[...], preferred_element_type=jnp.float32)
    y = y + b_ref[0:1, :]
    vt = lax.dot_general(w_ref[:, 2 * D:3 * D], hh,
                         (((0,), (1,)), ((), ())),
                         preferred_element_type=jnp.float32)
    vt = vt + bt_ref[2 * D:3 * D, 0:1]
    _split_proj(y, vt, q_ref, k_ref, s_ref, vt_ref)


def _final_body(o_ref, d_ref, sk_ref, h_ref, dt_ref):
    h_ref[...] = _assemble(o_ref, d_ref, sk_ref)
    d = jnp.sum(d_ref[...], axis=0)
    dt_ref[...] = jnp.concatenate(
        [jnp.broadcast_to(d[:, h:h + 1], (R_TC, CH)) for h in range(HEADS)], axis=1)


_row_spec = pl.BlockSpec((R_TC, D), lambda i: (i, 0))
_w_spec = pl.BlockSpec((F_IN, 4 * D), lambda i: (0, 0))
_b_spec = pl.BlockSpec((8, 4 * D), lambda i: (0, 0))
_bt_spec = pl.BlockSpec((4 * D, 8), lambda i: (0, 0))
_vt_spec = pl.BlockSpec((D, R_TC), lambda i: (0, i))
_occ_spec = pl.BlockSpec((NW, R_TC, 4), lambda i: (0, i, 0))

_proj_out = (
    jax.ShapeDtypeStruct((N_PAD, D), jnp.float32),
    jax.ShapeDtypeStruct((N_PAD, D), jnp.float32),
    jax.ShapeDtypeStruct((N_PAD, D), jnp.float32),
    jax.ShapeDtypeStruct((D, N_PAD), jnp.float32),
)
_proj_out_specs = (_row_spec, _row_spec, _row_spec, _vt_spec)

_proj_first = pl.pallas_call(
    _proj_first_body,
    grid=(_GRID,),
    in_specs=[_row_spec, _w_spec, _b_spec, _bt_spec],
    out_specs=_proj_out_specs,
    out_shape=_proj_out,
)

_proj_mid = pl.pallas_call(
    _proj_mid_body,
    grid=(_GRID,),
    in_specs=[_occ_spec, _occ_spec, _row_spec, _w_spec, _b_spec, _bt_spec],
    out_specs=_proj_out_specs,
    out_shape=_proj_out,
)

_final = pl.pallas_call(
    _final_body,
    grid=(_GRID,),
    in_specs=[_occ_spec, _occ_spec, _row_spec],
    out_specs=(_row_spec, _row_spec),
    out_shape=(
        jax.ShapeDtypeStruct((N_PAD, D), jnp.float32),
        jax.ShapeDtypeStruct((N_PAD, D), jnp.float32),
    ),
)


_MESH = plsc.VectorSubcoreMesh(core_axis_name="c", subcore_axis_name="s")
_SC_PARAMS = pltpu.CompilerParams(needs_layout_passes=False)
_SLAB = N_PAD * 4


def _edge_a_body(q_h, k_h, src_h, dst_h, ew_h, den_h,
                 sidx, didx, qr, kr, dent, est, sem):
    cid = lax.axis_index("c")
    sid = lax.axis_index("s")
    w = sid * 2 + cid
    z16 = jnp.zeros((16,), jnp.float32)
    lane = lax.iota(jnp.int32, 16)
    rowoff = lax.shift_right_logical(lane, 2)
    col = lax.bitwise_and(lane, 3)

    def zero(i, carry):
        dent[pl.ds(i * 16, 16)] = z16
        return carry

    lax.fori_loop(0, _SLAB // 16, zero, 0)

    tbase = w * NCH * C

    def chunk(g, carry):
        base = tbase + g * C
        pltpu.sync_copy(src_h.at[pl.ds(base, C)], sidx)
        pltpu.sync_copy(dst_h.at[pl.ds(base, C)], didx)
        d1 = pltpu.async_copy(q_h.at[didx], qr, sem)
        d2 = pltpu.async_copy(k_h.at[sidx], kr, sem)
        d1.wait()
        d2.wait()

        def group(g4, c2):
            e0 = g4 * 4
            av = z16
            for e in range(4):
                for h in range(HEADS):
                    p0 = qr[e0 + e, pl.ds(h * CH, 16)] * kr[e0 + e, pl.ds(h * CH, 16)]
                    p1 = qr[e0 + e, pl.ds(h * CH + 16, 16)] * kr[e0 + e, pl.ds(h * CH + 16, 16)]
                    s = jnp.sum(p0) + jnp.sum(p1)
                    av = jnp.where(lane == (4 * e + h), jnp.full((16,), s, jnp.float32), av)
            ev = jnp.exp(av * INV_SQRT_CH)
            est[pl.ds(16 * g4, 16)] = ev
            dst16 = plsc.load_gather(didx, [e0 + rowoff])
            plsc.addupdate_scatter(dent, [lax.shift_left(dst16, 2) + col], ev)
            return c2

        lax.fori_loop(0, C // 4, group, 0)
        pltpu.sync_copy(est, ew_h.at[pl.ds(base * 4, C * 4)])
        return carry

    lax.fori_loop(0, NCH, chunk, 0)
    pltpu.sync_copy(dent, den_h.at[pl.ds(w * _SLAB, _SLAB)])


_edge_a = pl.kernel(
    _edge_a_body,
    out_type=(
        jax.ShapeDtypeStruct((E_PAD * 4,), jnp.float32),
        jax.ShapeDtypeStruct((NW * _SLAB,), jnp.float32),
    ),
    mesh=_MESH,
    compiler_params=_SC_PARAMS,
    scratch_types=[
        pltpu.VMEM((C,), jnp.int32),
        pltpu.VMEM((C,), jnp.int32),
        pltpu.VMEM((C, D), jnp.float32),
        pltpu.VMEM((C, D), jnp.float32),
        pltpu.VMEM((_SLAB,), jnp.float32),
        pltpu.VMEM((C * 4,), jnp.float32),
        pltpu.SemaphoreType.DMA,
    ],
)


def _edge_b_body(vt_h, src_h, dst_h, ew_h, outc_h,
                 sidx, didx, ech, vt, outa):
    cid = lax.axis_index("c")
    sid = lax.axis_index("s")
    w = sid * 2 + cid
    z16 = jnp.zeros((16,), jnp.float32)
    lane = lax.iota(jnp.int32, 16)
    c8 = lax.bitwise_and(w, 1) * 4

    def zero(i, carry):
        outa[pl.ds(i * 16, 16)] = z16
        return carry

    lax.fori_loop(0, _SLAB // 16, zero, 0)
    pltpu.sync_copy(vt_h.at[pl.ds((w // 2) * 8, 8)], vt)

    def chunk(g, carry):
        base = g * C2
        pltpu.sync_copy(src_h.at[pl.ds(base, C2)], sidx)
        pltpu.sync_copy(dst_h.at[pl.ds(base, C2)], didx)
        pltpu.sync_copy(ew_h.at[pl.ds(base * 4, C2 * 4)], ech)

        def group(g16, c2):
            lane16 = g16 * 16 + lane
            src16 = plsc.load_gather(sidx, [lane16])
            dst16 = plsc.load_gather(didx, [lane16])
            ewbase = lax.shift_left(lane16, 2)
            dbase = lax.shift_left(dst16, 2)
            for cl in range(4):
                hc = lax.shift_right_logical(w * 4 + cl, 5)
                vv = plsc.load_gather(vt, [jnp.full((16,), 0, jnp.int32) + (c8 + cl), src16])
                ev16 = plsc.load_gather(ech, [ewbase + hc])
                plsc.addupdate_scatter(outa, [dbase + cl], vv * ev16)
            return c2

        lax.fori_loop(0, C2 // 16, group, 0)
        return carry

    lax.fori_loop(0, NCH2, chunk, 0)
    pltpu.sync_copy(outa, outc_h.at[pl.ds(w * _SLAB, _SLAB)])


_edge_b = pl.kernel(
    _edge_b_body,
    out_type=jax.ShapeDtypeStruct((NW * _SLAB,), jnp.float32),
    mesh=_MESH,
    compiler_params=_SC_PARAMS,
    scratch_types=[
        pltpu.VMEM((C2,), jnp.int32),
        pltpu.VMEM((C2,), jnp.int32),
        pltpu.VMEM((C2 * 4,), jnp.float32),
        pltpu.VMEM((8, N_PAD), jnp.float32),
        pltpu.VMEM((_SLAB,), jnp.float32),
    ],
)


def _norm_body(exp_h, dst_h, den_h, a_h, didx, dr, ech, ach, sem):
    cid = lax.axis_index("c")
    sid = lax.axis_index("s")
    w = sid * 2 + cid
    lane = lax.iota(jnp.int32, 16)
    rowoff = lax.shift_right_logical(lane, 2)
    col = lax.bitwise_and(lane, 3)
    tbase = w * NCH * C

    def chunk(g, carry):
        base = tbase + g * C
        pltpu.sync_copy(dst_h.at[pl.ds(base, C)], didx)
        pltpu.sync_copy(exp_h.at[pl.ds(base * 4, C * 4)], ech)
        pltpu.async_copy(den_h.at[didx], dr, sem).wait()

        def group(g4, c2):
            ev = ech[pl.ds(16 * g4, 16)]
            dv = plsc.load_gather(dr, [g4 * 4 + rowoff, lax.shift_left(col, 5)])
            ach[pl.ds(16 * g4, 16)] = ev / (dv + 1e-16)
            return c2

        lax.fori_loop(0, C // 4, group, 0)
        pltpu.sync_copy(ach, a_h.at[pl.ds(base * 4, C * 4)])
        return carry

    lax.fori_loop(0, NCH, chunk, 0)


_norm_kernel = pl.kernel(
    _norm_body,
    out_type=jax.ShapeDtypeStruct((E_PAD * 4,), jnp.float32),
    mesh=_MESH,
    compiler_params=_SC_PARAMS,
    scratch_types=[
        pltpu.VMEM((C,), jnp.int32),
        pltpu.VMEM((C, D), jnp.float32),
        pltpu.VMEM((C * 4,), jnp.float32),
        pltpu.VMEM((C * 4,), jnp.float32),
        pltpu.SemaphoreType.DMA,
    ],
)


def _build_edges(edge_index, batch):
    num_real = batch.shape[0]
    virtual_nodes = jnp.tile(jnp.arange(VIRT, dtype=jnp.int32), N_GRAPHS)
    batch_ext = jnp.concatenate([batch.astype(jnp.int32),
                                 jnp.tile(jnp.arange(N_GRAPHS, dtype=jnp.int32), VIRT)])
    b_sorted = jnp.sort(batch_ext)
    virt_edges = (num_real + b_sorted[:, None] * VIRT
                  + jnp.arange(VIRT, dtype=jnp.int32)[None, :]).reshape(-1)
    real_ids = jnp.arange(num_real, dtype=jnp.int32)
    src_edges = jnp.concatenate([real_ids, virt_edges])
    dst_edges = jnp.concatenate([virt_edges, real_ids])
    src = jnp.concatenate([edge_index[0].astype(jnp.int32), src_edges])
    dst = jnp.concatenate([edge_index[1].astype(jnp.int32), dst_edges])
    return src, dst, virtual_nodes


@jax.jit
def kernel(x, edge_index, batch, emb, Wq, bq, Wk, bk, Wv, bv, Wskip, bskip):
    src, dst, vn = _build_edges(edge_index, batch)
    pad_e = jnp.full((E_PAD - E_TOT,), DUMMY, dtype=jnp.int32)
    src_p = jnp.concatenate([src, pad_e])
    dst_p = jnp.concatenate([dst, pad_e])

    h0 = jnp.concatenate([
        x,
        jnp.take(emb, vn, axis=0),
        jnp.zeros((N_PAD - N_EXT, F_IN), jnp.float32),
    ], axis=0)

    Wcat = jnp.concatenate([Wq, Wk, Wv, Wskip], axis=2)
    bcat = jnp.concatenate([bq, bk, bv, bskip], axis=1)
    b8 = jnp.broadcast_to(bcat[:, None, :], (N_LAYERS, 8, 4 * D))
    bt8 = jnp.broadcast_to(bcat[:, :, None], (N_LAYERS, 4 * D, 8))

    q, k, sk, vT = _proj_first(h0, Wcat[0], b8[0], bt8[0])
    ew, den_p = _edge_a(q, k, src_p, dst_p)
    outc = _edge_b(vT, src_p, dst_p, ew)
    for l in range(1, N_LAYERS):
        q, k, sk_n, vT = _proj_mid(outc.reshape(NW, N_PAD, 4),
                                   den_p.reshape(NW, N_PAD, 4),
                                   sk, Wcat[l], b8[l], bt8[l])
        sk = sk_n
        ew, den_p = _edge_a(q, k, src_p, dst_p)
        outc = _edge_b(vT, src_p, dst_p, ew)

    h_out, den_tot = _final(outc.reshape(NW, N_PAD, 4),
                            den_p.reshape(NW, N_PAD, 4), sk)
    a_f = _norm_kernel(ew, dst_p, den_tot)
    return h_out[:N_NODES], a_f.reshape(E_PAD, HEADS)[:E_TOT]


# speedup vs baseline: 17.4239x; 1.2073x over previous
import functools
import math

import jax
import jax.numpy as jnp
from jax import lax
from jax.experimental import pallas as pl
from jax.experimental.pallas import tpu as pltpu
from jax.experimental.pallas import tpu_sc as plsc

N_NODES = 10000
N_EDGES = 320000
F_IN = 128
HEADS = 4
CH = 32
N_LAYERS = 4
VIRT = 4
N_GRAPHS = 8

D = HEADS * CH
N_EXT = N_NODES + VIRT * N_GRAPHS
N_PAD = 10240
DUMMY = N_PAD - 1
E_TOT = N_EDGES + (N_EXT * VIRT) + N_NODES
NW = 32
C = 128
NCH = -(-E_TOT // (NW * C))
E_PAD = NW * C * NCH
C2 = 1024
NCH2 = E_PAD // C2
INV_SQRT_CH = 1.0 / math.sqrt(float(CH))
R_TC = 256
_GRID = N_PAD // R_TC


def _split_proj(y, vt, q_ref, k_ref, s_ref, vt_ref):
    q_ref[...] = y[:, 0:D]
    k_ref[...] = y[:, D:2 * D]
    s_ref[...] = y[:, 3 * D:4 * D]
    vt_ref[...] = vt


def _proj_first_body(h_ref, w_ref, b_ref, bt_ref, q_ref, k_ref, s_ref, vt_ref):
    hh = h_ref[...]
    y = jnp.dot(hh, w_ref[...], preferred_element_type=jnp.float32)
    y = y + b_ref[0:1, :]
    vt = lax.dot_general(w_ref[:, 2 * D:3 * D], hh,
                         (((0,), (1,)), ((), ())),
                         preferred_element_type=jnp.float32)
    vt = vt + bt_ref[2 * D:3 * D, 0:1]
    _split_proj(y, vt, q_ref, k_ref, s_ref, vt_ref)


def _assemble(o_ref, d_ref, sk_ref):
    o = jnp.concatenate([o_ref[wi] for wi in range(NW)], axis=1)
    d = jnp.sum(d_ref[...], axis=0) + 1e-16
    parts = []
    for h in range(HEADS):
        dh = jnp.broadcast_to(d[:, h:h + 1], (R_TC, CH))
        parts.append(o[:, h * CH:(h + 1) * CH] / dh)
    return jnp.concatenate(parts, axis=1) + sk_ref[...]


def _proj_mid_body(o_ref, d_ref, sk_ref, w_ref, b_ref, bt_ref,
                   q_ref, k_ref, s_ref, vt_ref):
    hh = _assemble(o_ref, d_ref, sk_ref)
    y = jnp.dot(hh, w_ref[...], preferred_element_type=jnp.float32)
    y = y + b_ref[0:1, :]
    vt = lax.dot_general(w_ref[:, 2 * D:3 * D], hh,
                         (((0,), (1,)), ((), ())),
                         preferred_element_type=jnp.float32)
    vt = vt + bt_ref[2 * D:3 * D, 0:1]
    _split_proj(y, vt, q_ref, k_ref, s_ref, vt_ref)


def _final_body(o_ref, d_ref, sk_ref, h_ref, dt_ref):
    h_ref[...] = _assemble(o_ref, d_ref, sk_ref)
    d = jnp.sum(d_ref[...], axis=0)
    dt_ref[...] = jnp.concatenate(
        [jnp.broadcast_to(d[:, h:h + 1], (R_TC, CH)) for h in range(HEADS)], axis=1)


_row_spec = pl.BlockSpec((R_TC, D), lambda i: (i, 0))
_w_spec = pl.BlockSpec((F_IN, 4 * D), lambda i: (0, 0))
_b_spec = pl.BlockSpec((8, 4 * D), lambda i: (0, 0))
_bt_spec = pl.BlockSpec((4 * D, 8), lambda i: (0, 0))
_vt_spec = pl.BlockSpec((D, R_TC), lambda i: (0, i))
_occ_spec = pl.BlockSpec((NW, R_TC, 4), lambda i: (0, i, 0))

_proj_out = (
    jax.ShapeDtypeStruct((N_PAD, D), jnp.float32),
    jax.ShapeDtypeStruct((N_PAD, D), jnp.float32),
    jax.ShapeDtypeStruct((N_PAD, D), jnp.float32),
    jax.ShapeDtypeStruct((D, N_PAD), jnp.float32),
)
_proj_out_specs = (_row_spec, _row_spec, _row_spec, _vt_spec)

_proj_first = pl.pallas_call(
    _proj_first_body,
    grid=(_GRID,),
    in_specs=[_row_spec, _w_spec, _b_spec, _bt_spec],
    out_specs=_proj_out_specs,
    out_shape=_proj_out,
)

_proj_mid = pl.pallas_call(
    _proj_mid_body,
    grid=(_GRID,),
    in_specs=[_occ_spec, _occ_spec, _row_spec, _w_spec, _b_spec, _bt_spec],
    out_specs=_proj_out_specs,
    out_shape=_proj_out,
)

_final = pl.pallas_call(
    _final_body,
    grid=(_GRID,),
    in_specs=[_occ_spec, _occ_spec, _row_spec],
    out_specs=(_row_spec, _row_spec),
    out_shape=(
        jax.ShapeDtypeStruct((N_PAD, D), jnp.float32),
        jax.ShapeDtypeStruct((N_PAD, D), jnp.float32),
    ),
)


_MESH = plsc.VectorSubcoreMesh(core_axis_name="c", subcore_axis_name="s")
_SC_PARAMS = pltpu.CompilerParams(needs_layout_passes=False)
_SLAB = N_PAD * 4


def _edge_a_body(q_h, k_h, src_h, dst_h, ew_h, den_h,
                 sidx, didx, qr, kr, dent, est, sem):
    cid = lax.axis_index("c")
    sid = lax.axis_index("s")
    w = sid * 2 + cid
    z16 = jnp.zeros((16,), jnp.float32)
    lane = lax.iota(jnp.int32, 16)
    rowoff = lax.shift_right_logical(lane, 2)
    col = lax.bitwise_and(lane, 3)

    def zero(i, carry):
        dent[pl.ds(i * 16, 16)] = z16
        return carry

    lax.fori_loop(0, _SLAB // 16, zero, 0)

    tbase = w * NCH * C

    def chunk(g, carry):
        base = tbase + g * C
        pltpu.sync_copy(src_h.at[pl.ds(base, C)], sidx)
        pltpu.sync_copy(dst_h.at[pl.ds(base, C)], didx)
        d1 = pltpu.async_copy(q_h.at[didx], qr, sem)
        d2 = pltpu.async_copy(k_h.at[sidx], kr, sem)
        d1.wait()
        d2.wait()

        def group(g4, c2):
            e0 = g4 * 4
            av = z16
            for e in range(4):
                for h in range(HEADS):
                    p0 = qr[e0 + e, pl.ds(h * CH, 16)] * kr[e0 + e, pl.ds(h * CH, 16)]
                    p1 = qr[e0 + e, pl.ds(h * CH + 16, 16)] * kr[e0 + e, pl.ds(h * CH + 16, 16)]
                    s = jnp.sum(p0) + jnp.sum(p1)
                    av = jnp.where(lane == (4 * e + h), jnp.full((16,), s, jnp.float32), av)
            ev = jnp.exp(av * INV_SQRT_CH)
            est[pl.ds(16 * g4, 16)] = ev
            dst16 = plsc.load_gather(didx, [e0 + rowoff])
            plsc.addupdate_scatter(dent, [lax.shift_left(dst16, 2) + col], ev)
            return c2

        lax.fori_loop(0, C // 4, group, 0)
        pltpu.sync_copy(est, ew_h.at[pl.ds(base * 4, C * 4)])
        return carry

    lax.fori_loop(0, NCH, chunk, 0)
    pltpu.sync_copy(dent, den_h.at[pl.ds(w * _SLAB, _SLAB)])


_edge_a = pl.kernel(
    _edge_a_body,
    out_type=(
        jax.ShapeDtypeStruct((E_PAD * 4,), jnp.float32),
        jax.ShapeDtypeStruct((NW * _SLAB,), jnp.float32),
    ),
    mesh=_MESH,
    compiler_params=_SC_PARAMS,
    scratch_types=[
        pltpu.VMEM((C,), jnp.int32),
        pltpu.VMEM((C,), jnp.int32),
        pltpu.VMEM((C, D), jnp.float32),
        pltpu.VMEM((C, D), jnp.float32),
        pltpu.VMEM((_SLAB,), jnp.float32),
        pltpu.VMEM((C * 4,), jnp.float32),
        pltpu.SemaphoreType.DMA,
    ],
)


def _edge_b_body(vt_h, src_h, dst_h, ew_h, outc_h,
                 sidx, didx, ech, vt, outa):
    cid = lax.axis_index("c")
    sid = lax.axis_index("s")
    w = sid * 2 + cid
    z16 = jnp.zeros((16,), jnp.float32)
    lane = lax.iota(jnp.int32, 16)
    c8 = lax.bitwise_and(w, 1) * 4

    def zero(i, carry):
        outa[pl.ds(i * 16, 16)] = z16
        return carry

    lax.fori_loop(0, _SLAB // 16, zero, 0)
    pltpu.sync_copy(vt_h.at[pl.ds((w // 2) * 8, 8)], vt)

    def chunk(g, carry):
        base = g * C2
        pltpu.sync_copy(src_h.at[pl.ds(base, C2)], sidx)
        pltpu.sync_copy(dst_h.at[pl.ds(base, C2)], didx)
        pltpu.sync_copy(ew_h.at[pl.ds(base * 4, C2 * 4)], ech)

        def group(g16, c2):
            lane16 = g16 * 16 + lane
            src16 = plsc.load_gather(sidx, [lane16])
            dst16 = plsc.load_gather(didx, [lane16])
            ewbase = lax.shift_left(lane16, 2)
            dbase = lax.shift_left(dst16, 2)
            for cl in range(4):
                hc = lax.shift_right_logical(w * 4 + cl, 5)
                vv = plsc.load_gather(vt, [jnp.full((16,), 0, jnp.int32) + (c8 + cl), src16])
                ev16 = plsc.load_gather(ech, [ewbase + hc])
                plsc.addupdate_scatter(outa, [dbase + cl], vv * ev16)
            return c2

        lax.fori_loop(0, C2 // 16, group, 0)
        return carry

    lax.fori_loop(0, NCH2, chunk, 0)
    pltpu.sync_copy(outa, outc_h.at[pl.ds(w * _SLAB, _SLAB)])


_edge_b = pl.kernel(
    _edge_b_body,
    out_type=jax.ShapeDtypeStruct((NW * _SLAB,), jnp.float32),
    mesh=_MESH,
    compiler_params=_SC_PARAMS,
    scratch_types=[
        pltpu.VMEM((C2,), jnp.int32),
        pltpu.VMEM((C2,), jnp.int32),
        pltpu.VMEM((C2 * 4,), jnp.float32),
        pltpu.VMEM((8, N_PAD), jnp.float32),
        pltpu.VMEM((_SLAB,), jnp.float32),
    ],
)


def _norm_body(exp_h, dst_h, den_h, a_h, didx, dr, ech, ach, sem):
    cid = lax.axis_index("c")
    sid = lax.axis_index("s")
    w = sid * 2 + cid
    lane = lax.iota(jnp.int32, 16)
    rowoff = lax.shift_right_logical(lane, 2)
    col = lax.bitwise_and(lane, 3)
    tbase = w * NCH * C

    def chunk(g, carry):
        base = tbase + g * C
        pltpu.sync_copy(dst_h.at[pl.ds(base, C)], didx)
        pltpu.sync_copy(exp_h.at[pl.ds(base * 4, C * 4)], ech)
        pltpu.async_copy(den_h.at[didx], dr, sem).wait()

        def group(g4, c2):
            ev = ech[pl.ds(16 * g4, 16)]
            dv = plsc.load_gather(dr, [g4 * 4 + rowoff, lax.shift_left(col, 5)])
            ach[pl.ds(16 * g4, 16)] = ev / (dv + 1e-16)
            return c2

        lax.fori_loop(0, C // 4, group, 0)
        pltpu.sync_copy(ach, a_h.at[pl.ds(base * 4, C * 4)])
        return carry

    lax.fori_loop(0, NCH, chunk, 0)


_norm_kernel = pl.kernel(
    _norm_body,
    out_type=jax.ShapeDtypeStruct((E_PAD * 4,), jnp.float32),
    mesh=_MESH,
    compiler_params=_SC_PARAMS,
    scratch_types=[
        pltpu.VMEM((C,), jnp.int32),
        pltpu.VMEM((C, D), jnp.float32),
        pltpu.VMEM((C * 4,), jnp.float32),
        pltpu.VMEM((C * 4,), jnp.float32),
        pltpu.SemaphoreType.DMA,
    ],
)


def _build_edges(edge_index, batch):
    num_real = batch.shape[0]
    virtual_nodes = jnp.tile(jnp.arange(VIRT, dtype=jnp.int32), N_GRAPHS)
    batch_ext = jnp.concatenate([batch.astype(jnp.int32),
                                 jnp.tile(jnp.arange(N_GRAPHS, dtype=jnp.int32), VIRT)])
    b_sorted = jnp.sort(batch_ext)
    virt_edges = (num_real + b_sorted[:, None] * VIRT
                  + jnp.arange(VIRT, dtype=jnp.int32)[None, :]).reshape(-1)
    real_ids = jnp.arange(num_real, dtype=jnp.int32)
    src_edges = jnp.concatenate([real_ids, virt_edges])
    dst_edges = jnp.concatenate([virt_edges, real_ids])
    src = jnp.concatenate([edge_index[0].astype(jnp.int32), src_edges])
    dst = jnp.concatenate([edge_index[1].astype(jnp.int32), dst_edges])
    return src, dst, virtual_nodes


@jax.jit
def kernel(x, edge_index, batch, emb, Wq, bq, Wk, bk, Wv, bv, Wskip, bskip):
    src, dst, vn = _build_edges(edge_index, batch)
    pad_e = jnp.full((E_PAD - E_TOT,), DUMMY, dtype=jnp.int32)
    src_p = jnp.concatenate([src, pad_e])
    dst_p = jnp.concatenate([dst, pad_e])

    h0 = jnp.concatenate([
        x,
        jnp.take(emb, vn, axis=0),
        jnp.zeros((N_PAD - N_EXT, F_IN), jnp.float32),
    ], axis=0)

    Wcat = jnp.concatenate([Wq, Wk, Wv, Wskip], axis=2)
    bcat = jnp.concatenate([bq, bk, bv, bskip], axis=1)
    b8 = jnp.broadcast_to(bcat[:, None, :], (N_LAYERS, 8, 4 * D))
    bt8 = jnp.broadcast_to(bcat[:, :, None], (N_LAYERS, 4 * D, 8))

    q, k, sk, vT = _proj_first(h0, Wcat[0], b8[0], bt8[0])
    ew, den_p = _edge_a(q, k, src_p, dst_p)
    outc = _edge_b(vT, src_p, dst_p, ew)
    for l in range(1, N_LAYERS):
        q, k, sk_n, vT = _proj_mid(outc.reshape(NW, N_PAD, 4),
                                   den_p.reshape(NW, N_PAD, 4),
                                   sk, Wcat[l], b8[l], bt8[l])
        sk = sk_n
        ew, den_p = _edge_a(q, k, src_p, dst_p)
        outc = _edge_b(vT, src_p, dst_p, ew)

    h_out, den_tot = _final(outc.reshape(NW, N_PAD, 4),
                            den_p.reshape(NW, N_PAD, 4), sk)
    a_f = _norm_kernel(ew, dst_p, den_tot)
    return h_out[:N_NODES], a_f.reshape(E_PAD, HEADS)[:E_TOT]


# speedup vs baseline: 17.7859x; 1.0208x over previous
import functools
import math

import jax
import jax.numpy as jnp
from jax import lax
from jax.experimental import pallas as pl
from jax.experimental.pallas import tpu as pltpu
from jax.experimental.pallas import tpu_sc as plsc

N_NODES = 10000
N_EDGES = 320000
F_IN = 128
HEADS = 4
CH = 32
N_LAYERS = 4
VIRT = 4
N_GRAPHS = 8

D = HEADS * CH
N_EXT = N_NODES + VIRT * N_GRAPHS
N_PAD = 10240
DUMMY = N_PAD - 1
E_TOT = N_EDGES + (N_EXT * VIRT) + N_NODES
NW = 32
C = 128
NCH = -(-E_TOT // (NW * C))
E_PAD = NW * C * NCH
C2 = 1024
NCH2 = E_PAD // C2
INV_SQRT_CH = 1.0 / math.sqrt(float(CH))
R_TC = 256
_GRID = N_PAD // R_TC


def _split_proj(y, vt, q_ref, k_ref, s_ref, vt_ref):
    q_ref[...] = y[:, 0:D]
    k_ref[...] = y[:, D:2 * D]
    s_ref[...] = y[:, 3 * D:4 * D]
    vt_ref[...] = vt


def _proj_first_body(h_ref, w_ref, b_ref, bt_ref, q_ref, k_ref, s_ref, vt_ref):
    hh = h_ref[...]
    y = jnp.dot(hh, w_ref[...], preferred_element_type=jnp.float32)
    y = y + b_ref[0:1, :]
    vt = lax.dot_general(w_ref[:, 2 * D:3 * D], hh,
                         (((0,), (1,)), ((), ())),
                         preferred_element_type=jnp.float32)
    vt = vt + bt_ref[2 * D:3 * D, 0:1]
    _split_proj(y, vt, q_ref, k_ref, s_ref, vt_ref)


def _assemble(o_ref, d_ref, sk_ref):
    o = jnp.concatenate([o_ref[wi] for wi in range(NW)], axis=1)
    d = jnp.sum(d_ref[...], axis=0) + 1e-16
    parts = []
    for h in range(HEADS):
        dh = jnp.broadcast_to(d[:, h:h + 1], (R_TC, CH))
        parts.append(o[:, h * CH:(h + 1) * CH] / dh)
    return jnp.concatenate(parts, axis=1) + sk_ref[...]


def _proj_mid_body(o_ref, d_ref, sk_ref, w_ref, b_ref, bt_ref,
                   q_ref, k_ref, s_ref, vt_ref):
    hh = _assemble(o_ref, d_ref, sk_ref)
    y = jnp.dot(hh, w_ref[...], preferred_element_type=jnp.float32)
    y = y + b_ref[0:1, :]
    vt = lax.dot_general(w_ref[:, 2 * D:3 * D], hh,
                         (((0,), (1,)), ((), ())),
                         preferred_element_type=jnp.float32)
    vt = vt + bt_ref[2 * D:3 * D, 0:1]
    _split_proj(y, vt, q_ref, k_ref, s_ref, vt_ref)


def _final_body(o_ref, d_ref, sk_ref, h_ref, dt_ref):
    h_ref[...] = _assemble(o_ref, d_ref, sk_ref)
    d = jnp.sum(d_ref[...], axis=0)
    dt_ref[...] = jnp.concatenate(
        [jnp.broadcast_to(d[:, h:h + 1], (R_TC, CH)) for h in range(HEADS)], axis=1)


_row_spec = pl.BlockSpec((R_TC, D), lambda i: (i, 0))
_w_spec = pl.BlockSpec((F_IN, 4 * D), lambda i: (0, 0))
_b_spec = pl.BlockSpec((8, 4 * D), lambda i: (0, 0))
_bt_spec = pl.BlockSpec((4 * D, 8), lambda i: (0, 0))
_vt_spec = pl.BlockSpec((D, R_TC), lambda i: (0, i))
_occ_spec = pl.BlockSpec((NW, R_TC, 4), lambda i: (0, i, 0))

_proj_out = (
    jax.ShapeDtypeStruct((N_PAD, D), jnp.float32),
    jax.ShapeDtypeStruct((N_PAD, D), jnp.float32),
    jax.ShapeDtypeStruct((N_PAD, D), jnp.float32),
    jax.ShapeDtypeStruct((D, N_PAD), jnp.float32),
)
_proj_out_specs = (_row_spec, _row_spec, _row_spec, _vt_spec)

_proj_first = pl.pallas_call(
    _proj_first_body,
    grid=(_GRID,),
    in_specs=[_row_spec, _w_spec, _b_spec, _bt_spec],
    out_specs=_proj_out_specs,
    out_shape=_proj_out,
)

_proj_mid = pl.pallas_call(
    _proj_mid_body,
    grid=(_GRID,),
    in_specs=[_occ_spec, _occ_spec, _row_spec, _w_spec, _b_spec, _bt_spec],
    out_specs=_proj_out_specs,
    out_shape=_proj_out,
)

_final = pl.pallas_call(
    _final_body,
    grid=(_GRID,),
    in_specs=[_occ_spec, _occ_spec, _row_spec],
    out_specs=(_row_spec, _row_spec),
    out_shape=(
        jax.ShapeDtypeStruct((N_PAD, D), jnp.float32),
        jax.ShapeDtypeStruct((N_PAD, D), jnp.float32),
    ),
)


_MESH = plsc.VectorSubcoreMesh(core_axis_name="c", subcore_axis_name="s")
_SC_PARAMS = pltpu.CompilerParams(needs_layout_passes=False)
_SLAB = N_PAD * 4


def _edge_a_body(q_h, k_h, src_h, dst_h, ew_h, den_h,
                 sidx, didx, qr, kr, dent, est, sem):
    cid = lax.axis_index("c")
    sid = lax.axis_index("s")
    w = sid * 2 + cid
    z16 = jnp.zeros((16,), jnp.float32)
    lane = lax.iota(jnp.int32, 16)
    rowoff = lax.shift_right_logical(lane, 2)
    col = lax.bitwise_and(lane, 3)

    def zero(i, carry):
        dent[pl.ds(i * 16, 16)] = z16
        return carry

    lax.fori_loop(0, _SLAB // 16, zero, 0)

    tbase = w * NCH * C

    def chunk(g, carry):
        base = tbase + g * C
        pltpu.sync_copy(src_h.at[pl.ds(base, C)], sidx)
        pltpu.sync_copy(dst_h.at[pl.ds(base, C)], didx)
        d1 = pltpu.async_copy(q_h.at[didx], qr, sem)
        d2 = pltpu.async_copy(k_h.at[sidx], kr, sem)
        d1.wait()
        d2.wait()

        def group(g4, c2):
            e0 = g4 * 4
            av = z16
            for e in range(4):
                for h in range(HEADS):
                    p0 = qr[e0 + e, pl.ds(h * CH, 16)] * kr[e0 + e, pl.ds(h * CH, 16)]
                    p1 = qr[e0 + e, pl.ds(h * CH + 16, 16)] * kr[e0 + e, pl.ds(h * CH + 16, 16)]
                    s = jnp.sum(p0) + jnp.sum(p1)
                    av = jnp.where(lane == (4 * e + h), jnp.full((16,), s, jnp.float32), av)
            ev = jnp.exp(av * INV_SQRT_CH)
            est[pl.ds(16 * g4, 16)] = ev
            dst16 = plsc.load_gather(didx, [e0 + rowoff])
            plsc.addupdate_scatter(dent, [lax.shift_left(dst16, 2) + col], ev)
            return c2

        lax.fori_loop(0, C // 4, group, 0)
        pltpu.sync_copy(est, ew_h.at[pl.ds(base * 4, C * 4)])
        return carry

    lax.fori_loop(0, NCH, chunk, 0)
    pltpu.sync_copy(dent, den_h.at[pl.ds(w * _SLAB, _SLAB)])


_edge_a = pl.kernel(
    _edge_a_body,
    out_type=(
        jax.ShapeDtypeStruct((E_PAD * 4,), jnp.float32),
        jax.ShapeDtypeStruct((NW * _SLAB,), jnp.float32),
    ),
    mesh=_MESH,
    compiler_params=_SC_PARAMS,
    scratch_types=[
        pltpu.VMEM((C,), jnp.int32),
        pltpu.VMEM((C,), jnp.int32),
        pltpu.VMEM((C, D), jnp.float32),
        pltpu.VMEM((C, D), jnp.float32),
        pltpu.VMEM((_SLAB,), jnp.float32),
        pltpu.VMEM((C * 4,), jnp.float32),
        pltpu.SemaphoreType.DMA,
    ],
)


def _edge_b_body(vt_h, src_h, dst_h, ew_h, outc_h,
                 sidx, didx, ech, vt, outa):
    cid = lax.axis_index("c")
    sid = lax.axis_index("s")
    w = sid * 2 + cid
    z16 = jnp.zeros((16,), jnp.float32)
    lane = lax.iota(jnp.int32, 16)
    c8 = lax.bitwise_and(w, 1) * 4

    def zero(i, carry):
        outa[pl.ds(i * 16, 16)] = z16
        return carry

    lax.fori_loop(0, _SLAB // 16, zero, 0)
    pltpu.sync_copy(vt_h.at[pl.ds((w // 2) * 8, 8)], vt)

    def chunk(g, carry):
        base = g * C2
        pltpu.sync_copy(src_h.at[pl.ds(base, C2)], sidx)
        pltpu.sync_copy(dst_h.at[pl.ds(base, C2)], didx)
        pltpu.sync_copy(ew_h.at[pl.ds(base * 4, C2 * 4)], ech)

        def group(g16, c2):
            lane16 = g16 * 16 + lane
            src16 = plsc.load_gather(sidx, [lane16])
            dst16 = plsc.load_gather(didx, [lane16])
            dbase = lax.shift_left(dst16, 2)
            hc = lax.shift_right_logical(w, 3)
            ev16 = plsc.load_gather(ech, [lax.shift_left(lane16, 2) + hc])
            for cl in range(4):
                vv = plsc.load_gather(vt, [jnp.full((16,), 0, jnp.int32) + (c8 + cl), src16])
                plsc.addupdate_scatter(outa, [dbase + cl], vv * ev16)
            return c2

        lax.fori_loop(0, C2 // 16, group, 0)
        return carry

    lax.fori_loop(0, NCH2, chunk, 0)
    pltpu.sync_copy(outa, outc_h.at[pl.ds(w * _SLAB, _SLAB)])


_edge_b = pl.kernel(
    _edge_b_body,
    out_type=jax.ShapeDtypeStruct((NW * _SLAB,), jnp.float32),
    mesh=_MESH,
    compiler_params=_SC_PARAMS,
    scratch_types=[
        pltpu.VMEM((C2,), jnp.int32),
        pltpu.VMEM((C2,), jnp.int32),
        pltpu.VMEM((C2 * 4,), jnp.float32),
        pltpu.VMEM((8, N_PAD), jnp.float32),
        pltpu.VMEM((_SLAB,), jnp.float32),
    ],
)


def _norm_body(exp_h, dst_h, den_h, a_h, didx, dr, ech, ach, sem):
    cid = lax.axis_index("c")
    sid = lax.axis_index("s")
    w = sid * 2 + cid
    lane = lax.iota(jnp.int32, 16)
    rowoff = lax.shift_right_logical(lane, 2)
    col = lax.bitwise_and(lane, 3)
    tbase = w * NCH * C

    def chunk(g, carry):
        base = tbase + g * C
        pltpu.sync_copy(dst_h.at[pl.ds(base, C)], didx)
        pltpu.sync_copy(exp_h.at[pl.ds(base * 4, C * 4)], ech)
        pltpu.async_copy(den_h.at[didx], dr, sem).wait()

        def group(g4, c2):
            ev = ech[pl.ds(16 * g4, 16)]
            dv = plsc.load_gather(dr, [g4 * 4 + rowoff, lax.shift_left(col, 5)])
            ach[pl.ds(16 * g4, 16)] = ev / (dv + 1e-16)
            return c2

        lax.fori_loop(0, C // 4, group, 0)
        pltpu.sync_copy(ach, a_h.at[pl.ds(base * 4, C * 4)])
        return carry

    lax.fori_loop(0, NCH, chunk, 0)


_norm_kernel = pl.kernel(
    _norm_body,
    out_type=jax.ShapeDtypeStruct((E_PAD * 4,), jnp.float32),
    mesh=_MESH,
    compiler_params=_SC_PARAMS,
    scratch_types=[
        pltpu.VMEM((C,), jnp.int32),
        pltpu.VMEM((C, D), jnp.float32),
        pltpu.VMEM((C * 4,), jnp.float32),
        pltpu.VMEM((C * 4,), jnp.float32),
        pltpu.SemaphoreType.DMA,
    ],
)


def _build_edges(edge_index, batch):
    num_real = batch.shape[0]
    virtual_nodes = jnp.tile(jnp.arange(VIRT, dtype=jnp.int32), N_GRAPHS)
    batch_ext = jnp.concatenate([batch.astype(jnp.int32),
                                 jnp.tile(jnp.arange(N_GRAPHS, dtype=jnp.int32), VIRT)])
    b_sorted = jnp.sort(batch_ext)
    virt_edges = (num_real + b_sorted[:, None] * VIRT
                  + jnp.arange(VIRT, dtype=jnp.int32)[None, :]).reshape(-1)
    real_ids = jnp.arange(num_real, dtype=jnp.int32)
    src_edges = jnp.concatenate([real_ids, virt_edges])
    dst_edges = jnp.concatenate([virt_edges, real_ids])
    src = jnp.concatenate([edge_index[0].astype(jnp.int32), src_edges])
    dst = jnp.concatenate([edge_index[1].astype(jnp.int32), dst_edges])
    return src, dst, virtual_nodes


@jax.jit
def kernel(x, edge_index, batch, emb, Wq, bq, Wk, bk, Wv, bv, Wskip, bskip):
    src, dst, vn = _build_edges(edge_index, batch)
    pad_e = jnp.full((E_PAD - E_TOT,), DUMMY, dtype=jnp.int32)
    src_p = jnp.concatenate([src, pad_e])
    dst_p = jnp.concatenate([dst, pad_e])

    h0 = jnp.concatenate([
        x,
        jnp.take(emb, vn, axis=0),
        jnp.zeros((N_PAD - N_EXT, F_IN), jnp.float32),
    ], axis=0)

    Wcat = jnp.concatenate([Wq, Wk, Wv, Wskip], axis=2)
    bcat = jnp.concatenate([bq, bk, bv, bskip], axis=1)
    b8 = jnp.broadcast_to(bcat[:, None, :], (N_LAYERS, 8, 4 * D))
    bt8 = jnp.broadcast_to(bcat[:, :, None], (N_LAYERS, 4 * D, 8))

    q, k, sk, vT = _proj_first(h0, Wcat[0], b8[0], bt8[0])
    ew, den_p = _edge_a(q, k, src_p, dst_p)
    outc = _edge_b(vT, src_p, dst_p, ew)
    for l in range(1, N_LAYERS):
        q, k, sk_n, vT = _proj_mid(outc.reshape(NW, N_PAD, 4),
                                   den_p.reshape(NW, N_PAD, 4),
                                   sk, Wcat[l], b8[l], bt8[l])
        sk = sk_n
        ew, den_p = _edge_a(q, k, src_p, dst_p)
        outc = _edge_b(vT, src_p, dst_p, ew)

    h_out, den_tot = _final(outc.reshape(NW, N_PAD, 4),
                            den_p.reshape(NW, N_PAD, 4), sk)
    a_f = _norm_kernel(ew, dst_p, den_tot)
    return h_out[:N_NODES], a_f.reshape(E_PAD, HEADS)[:E_TOT]
